# baseline (speedup 1.0000x reference)
import functools

import jax
import jax.numpy as jnp
from jax import lax
from jax.experimental import pallas as pl
from jax.experimental.pallas import tpu as pltpu
from jax.experimental.pallas import tpu_sc as plsc

N = 2048
C = 1024
H = 4096
E = 8
BLK = 256
NG = (2 * N) // BLK + E
NP = NG * BLK
HB = 512
HG = H // HB
NW = 32
TPW = N // NW
CH = 32


def _router_body(x_ref, wg_ref, pos0_ref, pos1_ref, w0_ref, w1_ref,
                 bexp_ref, tot_ref, xb16_ref):
    x = x_ref[...]
    xb16_ref[...] = x.astype(jnp.bfloat16)
    wg = wg_ref[...]
    logits = lax.dot_general(x, wg, (((1,), (1,)), ((), ())),
                             preferred_element_type=jnp.float32)
    m = jnp.max(logits, axis=1, keepdims=True)
    ex = jnp.exp(logits - m)
    probs = ex / jnp.sum(ex, axis=1, keepdims=True)
    idx = lax.broadcasted_iota(jnp.int32, (N, E), 1)
    m1 = jnp.max(probs, axis=1, keepdims=True)
    a1 = jnp.min(jnp.where(probs == m1, idx, E), axis=1, keepdims=True)
    probs2 = jnp.where(idx == a1, -jnp.inf, probs)
    m2 = jnp.max(probs2, axis=1, keepdims=True)
    a2 = jnp.min(jnp.where(probs2 == m2, idx, E), axis=1, keepdims=True)
    wsum = m1 + m2
    w0_ref[...] = m1 / wsum
    w1_ref[...] = m2 / wsum

    A0 = (idx == a1).astype(jnp.int32)
    A1 = (idx == a2).astype(jnp.int32)
    inc0, inc1 = A0, A1
    k = 1
    while k < N:
        z = jnp.zeros((k, E), jnp.int32)
        inc0 = inc0 + jnp.concatenate([z, inc0[:-k]], axis=0)
        inc1 = inc1 + jnp.concatenate([z, inc1[:-k]], axis=0)
        k *= 2
    excl0 = inc0 - A0
    excl1 = inc1 - A1
    s0 = inc0[N - 1:N, :]
    counts = s0 + inc1[N - 1:N, :]
    nb = (counts + (BLK - 1)) >> 8
    cnb = nb
    k = 1
    while k < E:
        z = jnp.zeros((1, k), jnp.int32)
        cnb = cnb + jnp.concatenate([z, cnb[:, :-k]], axis=1)
        k *= 2
    base = (cnb - nb) * BLK
    total = cnb[:, E - 1:E]
    pos0_ref[...] = jnp.sum(A0 * (base + excl0), axis=1, keepdims=True)
    pos1_ref[...] = jnp.sum(A1 * (base + s0 + excl1), axis=1, keepdims=True)

    gidx = lax.broadcasted_iota(jnp.int32, (NG, E), 0)
    braw = jnp.sum((gidx >= cnb).astype(jnp.int32), axis=1, keepdims=True)
    last_e = jnp.sum((cnb < total).astype(jnp.int32), axis=1, keepdims=True)
    bexp_ref[...] = jnp.minimum(braw, last_e)
    tot_ref[...] = total


_router = pl.pallas_call(
    _router_body,
    out_shape=[
        jax.ShapeDtypeStruct((N, 1), jnp.int32),
        jax.ShapeDtypeStruct((N, 1), jnp.int32),
        jax.ShapeDtypeStruct((N, 1), jnp.float32),
        jax.ShapeDtypeStruct((N, 1), jnp.float32),
        jax.ShapeDtypeStruct((NG, 1), jnp.int32),
        jax.ShapeDtypeStruct((1, 1), jnp.int32),
        jax.ShapeDtypeStruct((N, C), jnp.bfloat16),
    ],
)


@functools.cache
def _make_dispatch():
    mesh = plsc.VectorSubcoreMesh(core_axis_name="c", subcore_axis_name="s")

    @functools.partial(
        pl.kernel,
        out_type=[jax.ShapeDtypeStruct((NP, C // 2), jnp.int32),
                  jax.ShapeDtypeStruct((NP,), jnp.float32)],
        mesh=mesh,
        scratch_types=[pltpu.VMEM((TPW, C // 2), jnp.int32),
                       pltpu.VMEM((TPW,), jnp.int32),
                       pltpu.VMEM((TPW,), jnp.int32),
                       pltpu.VMEM((TPW,), jnp.float32),
                       pltpu.VMEM((TPW,), jnp.float32),
                       pltpu.SemaphoreType.DMA],
    )
    def dispatch(x_hbm, pos0_hbm, pos1_hbm, w0_hbm, w1_hbm, xg_hbm, sw_hbm,
                 rows_v, i0_v, i1_v, a0_v, a1_v, sem):
        wid = lax.axis_index("c") * 16 + lax.axis_index("s")
        b = wid * TPW
        pltpu.sync_copy(x_hbm.at[pl.ds(b, TPW)], rows_v)
        pltpu.sync_copy(pos0_hbm.at[pl.ds(b, TPW)], i0_v)
        pltpu.sync_copy(pos1_hbm.at[pl.ds(b, TPW)], i1_v)
        pltpu.sync_copy(w0_hbm.at[pl.ds(b, TPW)], a0_v)
        pltpu.sync_copy(w1_hbm.at[pl.ds(b, TPW)], a1_v)
        pltpu.async_copy(rows_v, xg_hbm.at[i0_v], sem).wait()
        pltpu.async_copy(rows_v, xg_hbm.at[i1_v], sem).wait()
        pltpu.async_copy(a0_v, sw_hbm.at[i0_v], sem).wait()
        pltpu.async_copy(a1_v, sw_hbm.at[i1_v], sem).wait()

    return dispatch


def _ffn_body(bexp_ref, tot_ref, xg_ref, w1_ref, w3_ref, w2_ref, sw_ref,
              out_ref):
    hg = pl.program_id(0)
    g = pl.program_id(1)

    @pl.when(g < tot_ref[0])
    def _():
        sl = pl.ds(g * BLK, BLK)
        xb = xg_ref[sl, :]
        w1 = w1_ref[0].astype(jnp.bfloat16)
        w3 = w3_ref[0].astype(jnp.bfloat16)
        a = lax.dot_general(xb, w1, (((1,), (1,)), ((), ())),
                            preferred_element_type=jnp.float32)
        bpre = lax.dot_general(xb, w3, (((1,), (1,)), ((), ())),
                               preferred_element_type=jnp.float32)
        h = ((a / (1.0 + jnp.exp(-a))) * bpre).astype(jnp.bfloat16)
        part = lax.dot_general(h, w2_ref[0].astype(jnp.bfloat16),
                               (((1,), (1,)), ((), ())),
                               preferred_element_type=jnp.float32)

        @pl.when(hg == 0)
        def _():
            out_ref[sl, :] = part

        @pl.when(jnp.logical_and(hg > 0, hg < HG - 1))
        def _():
            out_ref[sl, :] = out_ref[sl, :] + part

        @pl.when(hg == HG - 1)
        def _():
            out_ref[sl, :] = (out_ref[sl, :] + part) * sw_ref[sl, :]


_ffn = pl.pallas_call(
    _ffn_body,
    grid_spec=pltpu.PrefetchScalarGridSpec(
        num_scalar_prefetch=2,
        grid=(HG, NG),
        in_specs=[
            pl.BlockSpec((NP, C), lambda hg, g, bexp, tot: (0, 0)),
            pl.BlockSpec((1, HB, C), lambda hg, g, bexp, tot: (bexp[g], hg, 0)),
            pl.BlockSpec((1, HB, C), lambda hg, g, bexp, tot: (bexp[g], hg, 0)),
            pl.BlockSpec((1, C, HB), lambda hg, g, bexp, tot: (bexp[g], 0, hg)),
            pl.BlockSpec((NP, 1), lambda hg, g, bexp, tot: (0, 0)),
        ],
        out_specs=pl.BlockSpec((NP, C), lambda hg, g, bexp, tot: (0, 0)),
    ),
    out_shape=jax.ShapeDtypeStruct((NP, C), jnp.float32),
    compiler_params=pltpu.CompilerParams(
        dimension_semantics=("arbitrary", "arbitrary")),
)


@functools.cache
def _make_combine():
    mesh = plsc.VectorSubcoreMesh(core_axis_name="c", subcore_axis_name="s")

    @functools.partial(
        pl.kernel,
        out_type=jax.ShapeDtypeStruct((N, C), jnp.float32),
        mesh=mesh,
        scratch_types=[pltpu.VMEM((CH,), jnp.int32),
                       pltpu.VMEM((CH,), jnp.int32),
                       pltpu.VMEM((CH, C), jnp.float32),
                       pltpu.VMEM((CH, C), jnp.float32),
                       pltpu.SemaphoreType.DMA],
    )
    def combine(yg_hbm, pos0_hbm, pos1_hbm, out_hbm, i0_v, i1_v, r0_v, r1_v,
                sem):
        wid = lax.axis_index("c") * 16 + lax.axis_index("s")
        for ci in range(TPW // CH):
            b = wid * TPW + ci * CH
            pltpu.sync_copy(pos0_hbm.at[pl.ds(b, CH)], i0_v)
            pltpu.sync_copy(pos1_hbm.at[pl.ds(b, CH)], i1_v)
            pltpu.async_copy(yg_hbm.at[i0_v], r0_v, sem).wait()
            pltpu.async_copy(yg_hbm.at[i1_v], r1_v, sem).wait()
            for i in range(CH):
                def add_body(j, _, i=i):
                    sl = pl.ds(j * 16, 16)
                    r0_v[i, sl] = r0_v[i, sl] + r1_v[i, sl]
                    return 0
                lax.fori_loop(0, C // 16, add_body, 0)
            pltpu.sync_copy(r0_v, out_hbm.at[pl.ds(b, CH)])

    return combine


def kernel(x, Wg, W1, W2, W3):
    Bb, Tt, Cc = x.shape
    xf = x.reshape(Tt, Cc)
    pos0, pos1, w0, w1, bexp, tot, xb16 = _router(xf, Wg)
    p0 = pos0.reshape(N)
    p1 = pos1.reshape(N)
    xpk = lax.bitcast_convert_type(xb16.reshape(N, C // 2, 2), jnp.int32)
    xg, sw = _make_dispatch()(xpk, p0, p1, w0.reshape(N), w1.reshape(N))
    xg_bf = lax.bitcast_convert_type(xg, jnp.bfloat16).reshape(NP, C)
    yg = _ffn(bexp.reshape(NG), tot.reshape(1), xg_bf, W1, W3, W2,
              sw.reshape(NP, 1))
    out = _make_combine()(yg, p0, p1)
    return out.reshape(Bb, Tt, Cc)


# speedup vs baseline: 1.1679x; 1.1679x over previous
import functools

import jax
import jax.numpy as jnp
from jax import lax
from jax.experimental import pallas as pl
from jax.experimental.pallas import tpu as pltpu
from jax.experimental.pallas import tpu_sc as plsc

N = 2048
C = 1024
H = 4096
E = 8
BLK = 256
NG = (2 * N + E * (BLK - 1)) // BLK
NP = NG * BLK
HB = 512
HG = H // HB
NW = 32
TPW = N // NW
CH = 32


def _router_body(x_ref, wg_ref, pos0_ref, pos1_ref, w0_ref, w1_ref,
                 cnb0_ref, xb16_ref):
    x = x_ref[...]
    xb16_ref[...] = x.astype(jnp.bfloat16)
    wg = wg_ref[...]
    logits = lax.dot_general(x, wg, (((1,), (1,)), ((), ())),
                             preferred_element_type=jnp.float32)
    m = jnp.max(logits, axis=1, keepdims=True)
    ex = jnp.exp(logits - m)
    probs = ex / jnp.sum(ex, axis=1, keepdims=True)
    idx = lax.broadcasted_iota(jnp.int32, (N, E), 1)
    m1 = jnp.max(probs, axis=1, keepdims=True)
    a1 = jnp.min(jnp.where(probs == m1, idx, E), axis=1, keepdims=True)
    probs2 = jnp.where(idx == a1, -jnp.inf, probs)
    m2 = jnp.max(probs2, axis=1, keepdims=True)
    a2 = jnp.min(jnp.where(probs2 == m2, idx, E), axis=1, keepdims=True)
    wsum = m1 + m2
    w0_ref[...] = m1 / wsum
    w1_ref[...] = m2 / wsum

    A0 = (idx == a1).astype(jnp.int32)
    A1 = (idx == a2).astype(jnp.int32)
    inc0, inc1 = A0, A1
    k = 1
    while k < N:
        z = jnp.zeros((k, E), jnp.int32)
        inc0 = inc0 + jnp.concatenate([z, inc0[:-k]], axis=0)
        inc1 = inc1 + jnp.concatenate([z, inc1[:-k]], axis=0)
        k *= 2
    excl0 = inc0 - A0
    excl1 = inc1 - A1
    s0 = inc0[N - 1:N, :]
    counts = s0 + inc1[N - 1:N, :]
    nb = (counts + (BLK - 1)) >> 8
    cnb = nb
    k = 1
    while k < E:
        z = jnp.zeros((1, k), jnp.int32)
        cnb = cnb + jnp.concatenate([z, cnb[:, :-k]], axis=1)
        k *= 2
    base = (cnb - nb) * BLK
    pos0_ref[...] = jnp.sum(A0 * (base + excl0), axis=1, keepdims=True)
    pos1_ref[...] = jnp.sum(A1 * (base + s0 + excl1), axis=1, keepdims=True)
    cnb0_ref[...] = jnp.concatenate([jnp.zeros((1, 1), jnp.int32), cnb],
                                    axis=1)


_router = pl.pallas_call(
    _router_body,
    out_shape=[
        jax.ShapeDtypeStruct((N, 1), jnp.int32),
        jax.ShapeDtypeStruct((N, 1), jnp.int32),
        jax.ShapeDtypeStruct((N, 1), jnp.float32),
        jax.ShapeDtypeStruct((N, 1), jnp.float32),
        jax.ShapeDtypeStruct((1, E + 1), jnp.int32),
        jax.ShapeDtypeStruct((N, C), jnp.bfloat16),
    ],
)


@functools.cache
def _make_dispatch():
    mesh = plsc.VectorSubcoreMesh(core_axis_name="c", subcore_axis_name="s")

    @functools.partial(
        pl.kernel,
        out_type=[jax.ShapeDtypeStruct((NP, C // 2), jnp.int32),
                  jax.ShapeDtypeStruct((NP,), jnp.float32)],
        mesh=mesh,
        scratch_types=[pltpu.VMEM((TPW, C // 2), jnp.int32),
                       pltpu.VMEM((TPW,), jnp.int32),
                       pltpu.VMEM((TPW,), jnp.int32),
                       pltpu.VMEM((TPW,), jnp.float32),
                       pltpu.VMEM((TPW,), jnp.float32),
                       pltpu.SemaphoreType.DMA],
    )
    def dispatch(x_hbm, pos0_hbm, pos1_hbm, w0_hbm, w1_hbm, xg_hbm, sw_hbm,
                 rows_v, i0_v, i1_v, a0_v, a1_v, sem):
        wid = lax.axis_index("c") * 16 + lax.axis_index("s")
        b = wid * TPW
        pltpu.sync_copy(x_hbm.at[pl.ds(b, TPW)], rows_v)
        pltpu.sync_copy(pos0_hbm.at[pl.ds(b, TPW)], i0_v)
        pltpu.sync_copy(pos1_hbm.at[pl.ds(b, TPW)], i1_v)
        pltpu.sync_copy(w0_hbm.at[pl.ds(b, TPW)], a0_v)
        pltpu.sync_copy(w1_hbm.at[pl.ds(b, TPW)], a1_v)
        pltpu.async_copy(rows_v, xg_hbm.at[i0_v], sem).wait()
        pltpu.async_copy(rows_v, xg_hbm.at[i1_v], sem).wait()
        pltpu.async_copy(a0_v, sw_hbm.at[i0_v], sem).wait()
        pltpu.async_copy(a1_v, sw_hbm.at[i1_v], sem).wait()

    return dispatch


def _ffn_body(cnb0_ref, xg_ref, w1_ref, w3_ref, w2_ref, sw_ref, out_ref):
    hg = pl.program_id(0)
    e = pl.program_id(1)
    g0 = cnb0_ref[e]
    g1 = cnb0_ref[e + 1]
    w1 = w1_ref[0].astype(jnp.bfloat16)
    w3 = w3_ref[0].astype(jnp.bfloat16)
    w2 = w2_ref[0].astype(jnp.bfloat16)

    def block_body(g, carry):
        sl = pl.ds(g * BLK, BLK)
        xb = xg_ref[sl, :]
        a = lax.dot_general(xb, w1, (((1,), (1,)), ((), ())),
                            preferred_element_type=jnp.float32)
        bpre = lax.dot_general(xb, w3, (((1,), (1,)), ((), ())),
                               preferred_element_type=jnp.float32)
        h = ((a / (1.0 + jnp.exp(-a))) * bpre).astype(jnp.bfloat16)
        part = lax.dot_general(h, w2, (((1,), (1,)), ((), ())),
                               preferred_element_type=jnp.float32)

        @pl.when(hg == 0)
        def _():
            out_ref[sl, :] = part

        @pl.when(jnp.logical_and(hg > 0, hg < HG - 1))
        def _():
            out_ref[sl, :] = out_ref[sl, :] + part

        @pl.when(hg == HG - 1)
        def _():
            out_ref[sl, :] = (out_ref[sl, :] + part) * sw_ref[sl, :]

        return carry

    lax.fori_loop(g0, g1, block_body, 0)


_ffn = pl.pallas_call(
    _ffn_body,
    grid_spec=pltpu.PrefetchScalarGridSpec(
        num_scalar_prefetch=1,
        grid=(HG, E),
        in_specs=[
            pl.BlockSpec((NP, C), lambda hg, e, cnb0: (0, 0)),
            pl.BlockSpec((1, HB, C), lambda hg, e, cnb0: (e, hg, 0)),
            pl.BlockSpec((1, HB, C), lambda hg, e, cnb0: (e, hg, 0)),
            pl.BlockSpec((1, C, HB), lambda hg, e, cnb0: (e, 0, hg)),
            pl.BlockSpec((NP, 1), lambda hg, e, cnb0: (0, 0)),
        ],
        out_specs=pl.BlockSpec((NP, C), lambda hg, e, cnb0: (0, 0)),
    ),
    out_shape=jax.ShapeDtypeStruct((NP, C), jnp.float32),
    compiler_params=pltpu.CompilerParams(
        dimension_semantics=("arbitrary", "arbitrary")),
)


@functools.cache
def _make_combine():
    mesh = plsc.VectorSubcoreMesh(core_axis_name="c", subcore_axis_name="s")

    @functools.partial(
        pl.kernel,
        out_type=jax.ShapeDtypeStruct((N, C), jnp.float32),
        mesh=mesh,
        scratch_types=[pltpu.VMEM((CH,), jnp.int32),
                       pltpu.VMEM((CH,), jnp.int32),
                       pltpu.VMEM((CH, C), jnp.float32),
                       pltpu.VMEM((CH, C), jnp.float32),
                       pltpu.SemaphoreType.DMA],
    )
    def combine(yg_hbm, pos0_hbm, pos1_hbm, out_hbm, i0_v, i1_v, r0_v, r1_v,
                sem):
        wid = lax.axis_index("c") * 16 + lax.axis_index("s")
        for ci in range(TPW // CH):
            b = wid * TPW + ci * CH
            pltpu.sync_copy(pos0_hbm.at[pl.ds(b, CH)], i0_v)
            pltpu.sync_copy(pos1_hbm.at[pl.ds(b, CH)], i1_v)
            pltpu.async_copy(yg_hbm.at[i0_v], r0_v, sem).wait()
            pltpu.async_copy(yg_hbm.at[i1_v], r1_v, sem).wait()
            for i in range(CH):
                def add_body(j, _, i=i):
                    sl = pl.ds(j * 16, 16)
                    r0_v[i, sl] = r0_v[i, sl] + r1_v[i, sl]
                    return 0
                lax.fori_loop(0, C // 16, add_body, 0)
            pltpu.sync_copy(r0_v, out_hbm.at[pl.ds(b, CH)])

    return combine


def kernel(x, Wg, W1, W2, W3):
    Bb, Tt, Cc = x.shape
    xf = x.reshape(Tt, Cc)
    pos0, pos1, w0, w1, cnb0, xb16 = _router(xf, Wg)
    p0 = pos0.reshape(N)
    p1 = pos1.reshape(N)
    xpk = lax.bitcast_convert_type(xb16.reshape(N, C // 2, 2), jnp.int32)
    xg, sw = _make_dispatch()(xpk, p0, p1, w0.reshape(N), w1.reshape(N))
    xg_bf = lax.bitcast_convert_type(xg, jnp.bfloat16).reshape(NP, C)
    yg = _ffn(cnb0.reshape(E + 1), xg_bf, W1, W3, W2, sw.reshape(NP, 1))
    out = _make_combine()(yg, p0, p1)
    return out.reshape(Bb, Tt, Cc)


# speedup vs baseline: 1.6941x; 1.4505x over previous
import functools

import jax
import jax.numpy as jnp
from jax import lax
from jax.experimental import pallas as pl
from jax.experimental.pallas import tpu as pltpu
from jax.experimental.pallas import tpu_sc as plsc

N = 2048
C = 1024
H = 4096
E = 8
BLK = 256
NG = (2 * N + E * (BLK - 1)) // BLK
NP = NG * BLK
HB = 512
HG = H // HB
NW = 32
TPW = N // NW
CH = 32


def _router_body(x_ref, wg_ref, pos0_ref, pos1_ref, w0_ref, w1_ref,
                 cnb0_ref, xb16_ref):
    x = x_ref[...]
    blo = lax.bitcast_convert_type(
        x[:, :C // 2].astype(jnp.bfloat16).astype(jnp.float32), jnp.int32)
    bhi = lax.bitcast_convert_type(
        x[:, C // 2:].astype(jnp.bfloat16).astype(jnp.float32), jnp.int32)
    xb16_ref[...] = (bhi & jnp.int32(-65536)) | lax.shift_right_logical(
        blo, 16)
    wg = wg_ref[...]
    logits = lax.dot_general(x, wg, (((1,), (1,)), ((), ())),
                             preferred_element_type=jnp.float32)
    m = jnp.max(logits, axis=1, keepdims=True)
    ex = jnp.exp(logits - m)
    probs = ex / jnp.sum(ex, axis=1, keepdims=True)
    idx = lax.broadcasted_iota(jnp.int32, (N, E), 1)
    m1 = jnp.max(probs, axis=1, keepdims=True)
    a1 = jnp.min(jnp.where(probs == m1, idx, E), axis=1, keepdims=True)
    probs2 = jnp.where(idx == a1, -jnp.inf, probs)
    m2 = jnp.max(probs2, axis=1, keepdims=True)
    a2 = jnp.min(jnp.where(probs2 == m2, idx, E), axis=1, keepdims=True)
    wsum = m1 + m2
    w0_ref[...] = m1 / wsum
    w1_ref[...] = m2 / wsum

    A0 = (idx == a1).astype(jnp.int32)
    A1 = (idx == a2).astype(jnp.int32)
    inc0, inc1 = A0, A1
    k = 1
    while k < N:
        z = jnp.zeros((k, E), jnp.int32)
        inc0 = inc0 + jnp.concatenate([z, inc0[:-k]], axis=0)
        inc1 = inc1 + jnp.concatenate([z, inc1[:-k]], axis=0)
        k *= 2
    excl0 = inc0 - A0
    excl1 = inc1 - A1
    s0 = inc0[N - 1:N, :]
    counts = s0 + inc1[N - 1:N, :]
    nb = (counts + (BLK - 1)) >> 8
    cnb = nb
    k = 1
    while k < E:
        z = jnp.zeros((1, k), jnp.int32)
        cnb = cnb + jnp.concatenate([z, cnb[:, :-k]], axis=1)
        k *= 2
    base = (cnb - nb) * BLK
    pos0_ref[...] = jnp.sum(A0 * (base + excl0), axis=1, keepdims=True)
    pos1_ref[...] = jnp.sum(A1 * (base + s0 + excl1), axis=1, keepdims=True)
    cnb0_ref[...] = jnp.concatenate([jnp.zeros((1, 1), jnp.int32), cnb],
                                    axis=1)


_router = pl.pallas_call(
    _router_body,
    out_shape=[
        jax.ShapeDtypeStruct((N, 1), jnp.int32),
        jax.ShapeDtypeStruct((N, 1), jnp.int32),
        jax.ShapeDtypeStruct((N, 1), jnp.float32),
        jax.ShapeDtypeStruct((N, 1), jnp.float32),
        jax.ShapeDtypeStruct((1, E + 1), jnp.int32),
        jax.ShapeDtypeStruct((N, C // 2), jnp.int32),
    ],
)


@functools.cache
def _make_dispatch():
    mesh = plsc.VectorSubcoreMesh(core_axis_name="c", subcore_axis_name="s")

    @functools.partial(
        pl.kernel,
        out_type=[jax.ShapeDtypeStruct((NP, C // 2), jnp.int32),
                  jax.ShapeDtypeStruct((NP,), jnp.float32)],
        mesh=mesh,
        scratch_types=[pltpu.VMEM((TPW, C // 2), jnp.int32),
                       pltpu.VMEM((TPW,), jnp.int32),
                       pltpu.VMEM((TPW,), jnp.int32),
                       pltpu.VMEM((TPW,), jnp.float32),
                       pltpu.VMEM((TPW,), jnp.float32),
                       pltpu.SemaphoreType.DMA],
    )
    def dispatch(x_hbm, pos0_hbm, pos1_hbm, w0_hbm, w1_hbm, xg_hbm, sw_hbm,
                 rows_v, i0_v, i1_v, a0_v, a1_v, sem):
        wid = lax.axis_index("c") * 16 + lax.axis_index("s")
        b = wid * TPW
        pltpu.sync_copy(x_hbm.at[pl.ds(b, TPW)], rows_v)
        pltpu.sync_copy(pos0_hbm.at[pl.ds(b, TPW)], i0_v)
        pltpu.sync_copy(pos1_hbm.at[pl.ds(b, TPW)], i1_v)
        pltpu.sync_copy(w0_hbm.at[pl.ds(b, TPW)], a0_v)
        pltpu.sync_copy(w1_hbm.at[pl.ds(b, TPW)], a1_v)
        pltpu.async_copy(rows_v, xg_hbm.at[i0_v], sem).wait()
        pltpu.async_copy(rows_v, xg_hbm.at[i1_v], sem).wait()
        pltpu.async_copy(a0_v, sw_hbm.at[i0_v], sem).wait()
        pltpu.async_copy(a1_v, sw_hbm.at[i1_v], sem).wait()

    return dispatch


def _ffn_body(cnb0_ref, xg_ref, w1_ref, w3_ref, w2_ref, sw_ref, out_ref):
    hg = pl.program_id(0)
    e = pl.program_id(1)
    g0 = cnb0_ref[e]
    g1 = cnb0_ref[e + 1]
    w1 = w1_ref[0].astype(jnp.bfloat16)
    w3 = w3_ref[0].astype(jnp.bfloat16)
    w2 = w2_ref[0].astype(jnp.bfloat16)

    def block_body(g, carry):
        sl = pl.ds(g * BLK, BLK)
        pk = xg_ref[sl, :]
        xlo = lax.bitcast_convert_type(lax.shift_left(pk, 16),
                                       jnp.float32).astype(jnp.bfloat16)
        xhi = lax.bitcast_convert_type(pk & jnp.int32(-65536),
                                       jnp.float32).astype(jnp.bfloat16)
        xb = jnp.concatenate([xlo, xhi], axis=1)
        a = lax.dot_general(xb, w1, (((1,), (1,)), ((), ())),
                            preferred_element_type=jnp.float32)
        bpre = lax.dot_general(xb, w3, (((1,), (1,)), ((), ())),
                               preferred_element_type=jnp.float32)
        h = ((a / (1.0 + jnp.exp(-a))) * bpre).astype(jnp.bfloat16)
        part = lax.dot_general(h, w2, (((1,), (1,)), ((), ())),
                               preferred_element_type=jnp.float32)

        @pl.when(hg == 0)
        def _():
            out_ref[sl, :] = part

        @pl.when(jnp.logical_and(hg > 0, hg < HG - 1))
        def _():
            out_ref[sl, :] = out_ref[sl, :] + part

        @pl.when(hg == HG - 1)
        def _():
            out_ref[sl, :] = (out_ref[sl, :] + part) * sw_ref[sl, :]

        return carry

    lax.fori_loop(g0, g1, block_body, 0)


_ffn = pl.pallas_call(
    _ffn_body,
    grid_spec=pltpu.PrefetchScalarGridSpec(
        num_scalar_prefetch=1,
        grid=(HG, E),
        in_specs=[
            pl.BlockSpec((NP, C // 2), lambda hg, e, cnb0: (0, 0)),
            pl.BlockSpec((1, HB, C), lambda hg, e, cnb0: (e, hg, 0)),
            pl.BlockSpec((1, HB, C), lambda hg, e, cnb0: (e, hg, 0)),
            pl.BlockSpec((1, C, HB), lambda hg, e, cnb0: (e, 0, hg)),
            pl.BlockSpec((NP, 1), lambda hg, e, cnb0: (0, 0)),
        ],
        out_specs=pl.BlockSpec((NP, C), lambda hg, e, cnb0: (0, 0)),
    ),
    out_shape=jax.ShapeDtypeStruct((NP, C), jnp.float32),
    compiler_params=pltpu.CompilerParams(
        dimension_semantics=("arbitrary", "arbitrary")),
)


@functools.cache
def _make_combine():
    mesh = plsc.VectorSubcoreMesh(core_axis_name="c", subcore_axis_name="s")

    @functools.partial(
        pl.kernel,
        out_type=jax.ShapeDtypeStruct((N, C), jnp.float32),
        mesh=mesh,
        scratch_types=[pltpu.VMEM((CH,), jnp.int32),
                       pltpu.VMEM((CH,), jnp.int32),
                       pltpu.VMEM((CH, C), jnp.float32),
                       pltpu.VMEM((CH, C), jnp.float32),
                       pltpu.SemaphoreType.DMA],
    )
    def combine(yg_hbm, pos0_hbm, pos1_hbm, out_hbm, i0_v, i1_v, r0_v, r1_v,
                sem):
        wid = lax.axis_index("c") * 16 + lax.axis_index("s")
        for ci in range(TPW // CH):
            b = wid * TPW + ci * CH
            pltpu.sync_copy(pos0_hbm.at[pl.ds(b, CH)], i0_v)
            pltpu.sync_copy(pos1_hbm.at[pl.ds(b, CH)], i1_v)
            pltpu.async_copy(yg_hbm.at[i0_v], r0_v, sem).wait()
            pltpu.async_copy(yg_hbm.at[i1_v], r1_v, sem).wait()
            for i in range(CH):
                def add_body(j, _, i=i):
                    sl = pl.ds(j * 16, 16)
                    r0_v[i, sl] = r0_v[i, sl] + r1_v[i, sl]
                    return 0
                lax.fori_loop(0, C // 16, add_body, 0)
            pltpu.sync_copy(r0_v, out_hbm.at[pl.ds(b, CH)])

    return combine


def kernel(x, Wg, W1, W2, W3):
    Bb, Tt, Cc = x.shape
    xf = x.reshape(Tt, Cc)
    pos0, pos1, w0, w1, cnb0, xpk = _router(xf, Wg)
    p0 = pos0.reshape(N)
    p1 = pos1.reshape(N)
    xg, sw = _make_dispatch()(xpk, p0, p1, w0.reshape(N), w1.reshape(N))
    yg = _ffn(cnb0.reshape(E + 1), xg, W1, W3, W2, sw.reshape(NP, 1))
    out = _make_combine()(yg, p0, p1)
    return out.reshape(Bb, Tt, Cc)
